# baseline (speedup 1.0000x reference)
import functools

import jax
import jax.numpy as jnp
from jax import lax
from jax.experimental import pallas as pl
from jax.experimental.pallas import tpu as pltpu
from jax.experimental.pallas import tpu_sc as plsc

MARGIN = 1.0
EPS = 1e-6

D = 512
B = 16384
L = 16
NW = 32
BPW = B // NW
C = 16
NBUF = 2
NCHUNK = BPW // C
DV = D // L
PROWS = BPW // 8

_mesh = plsc.VectorSubcoreMesh(core_axis_name="c", subcore_axis_name="s")


@functools.partial(
    pl.kernel,
    mesh=_mesh,
    out_type=[
        jax.ShapeDtypeStruct((NW * PROWS, 128), jnp.float32),
        jax.ShapeDtypeStruct((NW * PROWS, 128), jnp.float32),
    ],
    scratch_types=[
        pltpu.VMEM((3 * BPW,), jnp.int32),
        *([pltpu.VMEM((3 * C, D), jnp.float32)] * 2),
        pltpu.VMEM((PROWS, 128), jnp.float32),
        pltpu.VMEM((PROWS, 128), jnp.float32),
        *([pltpu.SemaphoreType.DMA] * 2),
    ],
)
def _sc_ssd(table_hbm, idx_hbm, outp_hbm, outn_hbm,
            idx_v, r0, r1, ssdp_v, ssdn_v, sem0, sem1):
    wid = lax.axis_index("s") * 2 + lax.axis_index("c")
    pltpu.sync_copy(idx_hbm.at[pl.ds(wid * 3 * BPW, 3 * BPW)], idx_v)

    slots = ((r0, sem0), (r1, sem1))

    def issue(g, slot):
        rbuf, sem = slot
        pltpu.async_copy(table_hbm.at[idx_v.at[pl.ds(g * 3 * C, 3 * C)]],
                         rbuf, sem)

    def wait_slot(slot):
        rbuf, sem = slot
        pltpu.make_async_copy(table_hbm.at[pl.ds(0, 3 * C)], rbuf, sem).wait()

    def compute(g, slot):
        rbuf, _ = slot
        base = g * C

        def trip_body(i, carry2):
            accp = jnp.zeros((L,), jnp.float32)
            accn = jnp.zeros((L,), jnp.float32)
            for j in range(DV):
                a = rbuf[i, pl.ds(j * L, L)]
                p = rbuf[C + i, pl.ds(j * L, L)]
                n = rbuf[2 * C + i, pl.ds(j * L, L)]
                t = a + EPS
                dp = t - p
                dn = t - n
                accp = accp + dp * dp
                accn = accn + dn * dn
            t2 = base + i
            row = t2 // 8
            col = (t2 % 8) * L
            ssdp_v[row, pl.ds(col, L)] = accp
            ssdn_v[row, pl.ds(col, L)] = accn
            return carry2

        lax.fori_loop(0, C, trip_body, 0)

    for s in range(NBUF):
        issue(s, slots[s])

    def body4(gg, carry):
        g0 = NBUF * gg
        for s in range(NBUF):
            g = g0 + s
            wait_slot(slots[s])
            compute(g, slots[s])

            @pl.when(g + NBUF < NCHUNK)
            def _():
                issue(g + NBUF, slots[s])

        return carry

    lax.fori_loop(0, NCHUNK // NBUF, body4, 0)

    pltpu.sync_copy(ssdp_v, outp_hbm.at[pl.ds(wid * PROWS, PROWS)])
    pltpu.sync_copy(ssdn_v, outn_hbm.at[pl.ds(wid * PROWS, PROWS)])


def _tc_finish_body(pp_ref, pn_ref, out_ref):
    lanes = lax.broadcasted_iota(jnp.int32, (128, 8), 0) // L
    groups = lax.broadcasted_iota(jnp.int32, (128, 8), 1)
    g_mat = (lanes == groups).astype(jnp.float32)
    sp = jnp.dot(pp_ref[...], g_mat, preferred_element_type=jnp.float32)
    sn = jnp.dot(pn_ref[...], g_mat, preferred_element_type=jnp.float32)
    dp = sp * lax.rsqrt(jnp.maximum(sp, 1e-30))
    dn = sn * lax.rsqrt(jnp.maximum(sn, 1e-30))
    losses = jnp.maximum(dp - dn + MARGIN, 0.0)
    out_ref[...] = jnp.sum(losses).reshape(1, 1) / B


_tc_finish = pl.pallas_call(
    _tc_finish_body,
    out_shape=jax.ShapeDtypeStruct((1, 1), jnp.float32),
)


def kernel(outputs, anchors, positives, negatives):
    idx = jnp.stack(
        [anchors.astype(jnp.int32).reshape(NW, NCHUNK, C),
         positives.astype(jnp.int32).reshape(NW, NCHUNK, C),
         negatives.astype(jnp.int32).reshape(NW, NCHUNK, C)],
        axis=2,
    ).reshape(NW * NCHUNK * 3 * C)
    pp, pn = _sc_ssd(outputs, idx)
    res = _tc_finish(pp, pn)
    return res[0, 0]
